# baseline (speedup 1.0000x reference)
import functools

import jax
import jax.numpy as jnp
import numpy as np
from jax import lax
from jax.experimental import pallas as pl
from jax.experimental.pallas import tpu as pltpu
from jax.experimental.pallas import tpu_sc as plsc

RR = 128
CC = 128
HH = 32
EIN = 16
EHID = 64
NOUT = 3
VV = 6 * (RR // 4) ** 2
EE = VV * 16

NW = 32
EPW = EE // NW
CH = 128
NCH = EPW // CH
VS = VV // 16

@functools.cache
def _sc_gather_call():
    return pl.kernel(
        _sc_gather_body,
        out_type=jax.ShapeDtypeStruct((EE, HH), jnp.float32),
        mesh=plsc.VectorSubcoreMesh(core_axis_name="c", subcore_axis_name="s"),
        scratch_types=[
            pltpu.VMEM((EPW,), jnp.int32),
            pltpu.VMEM((EPW, HH), jnp.float32),
            pltpu.SemaphoreType.DMA,
        ],
        compiler_params=pltpu.CompilerParams(use_tc_tiling_on_sc=False),
    )


def _sc_gather_body(node_hbm, src_hbm, out_hbm, idx_v, rows_v, sem):
    wid = lax.axis_index("s") * 2 + lax.axis_index("c")
    base = wid * EPW
    pltpu.sync_copy(src_hbm.at[pl.ds(base, EPW)], idx_v)
    cps = []
    for j in range(NCH):
        cps.append(pltpu.async_copy(
            node_hbm.at[idx_v.at[pl.ds(j * CH, CH)]],
            rows_v.at[pl.ds(j * CH, CH)], sem))
    for cp in cps:
        cp.wait()
    pltpu.sync_copy(rows_v, out_hbm.at[pl.ds(base, EPW)])


@functools.cache
def _sc_scatter_call():
    return pl.kernel(
        _sc_scatter_body,
        out_type=jax.ShapeDtypeStruct((2 * VV, HH), jnp.float32),
        mesh=plsc.VectorSubcoreMesh(core_axis_name="c", subcore_axis_name="s"),
        scratch_types=[
            pltpu.VMEM((NCH, CH), jnp.int32),
            pltpu.VMEM((EPW, HH), jnp.float32),
            pltpu.VMEM_SHARED((VV, HH), jnp.float32),
        ],
        compiler_params=pltpu.CompilerParams(use_tc_tiling_on_sc=False),
    )


def _sc_scatter_body(msg_hbm, dst_hbm, zeros_hbm, out_hbm, idx_v, msg_v, acc):
    cid = lax.axis_index("c")
    sid = lax.axis_index("s")
    wid = sid * 2 + cid
    pltpu.sync_copy(zeros_hbm.at[pl.ds(sid * VS, VS)], acc.at[pl.ds(sid * VS, VS)])
    plsc.subcore_barrier()
    pltpu.sync_copy(dst_hbm.at[wid], idx_v)
    pltpu.sync_copy(msg_hbm.at[pl.ds(wid * EPW, EPW)], msg_v)
    for j in range(NCH):
        pltpu.sync_copy(msg_v.at[pl.ds(j * CH, CH)], acc.at[idx_v.at[j]], add=True)
    plsc.subcore_barrier()
    pltpu.sync_copy(acc.at[pl.ds(sid * VS, VS)],
                    out_hbm.at[pl.ds(cid * VV + sid * VS, VS)])


def _enc_body(x_ref, pm_ref, w1_ref, b1_ref, w2_ref, b2_ref, o_ref):
    x = x_ref[0]
    pooled = jnp.dot(pm_ref[...], x, preferred_element_type=jnp.float32)
    h = jnp.maximum(
        jnp.dot(pooled, w1_ref[...], preferred_element_type=jnp.float32)
        + b1_ref[...], 0.0)
    o_ref[0] = jnp.dot(h, w2_ref[...], preferred_element_type=jnp.float32) + b2_ref[...]


def _encoder(x3, pm, w1t, b1, w2t, b2):
    ncell = x3.shape[0]
    return pl.pallas_call(
        _enc_body,
        grid=(ncell,),
        in_specs=[
            pl.BlockSpec((1, 4 * RR, CC), lambda i: (i, 0, 0)),
            pl.BlockSpec((HH, 4 * RR), lambda i: (0, 0)),
            pl.BlockSpec((CC, HH), lambda i: (0, 0)),
            pl.BlockSpec((1, HH), lambda i: (0, 0)),
            pl.BlockSpec((HH, HH), lambda i: (0, 0)),
            pl.BlockSpec((1, HH), lambda i: (0, 0)),
        ],
        out_specs=pl.BlockSpec((1, HH, HH), lambda i: (i, 0, 0)),
        out_shape=jax.ShapeDtypeStruct((ncell, HH, HH), jnp.float32),
    )(x3, pm, w1t, b1, w2t, b2)


def _mlp_body(x_ref, w1_ref, b1_ref, w2_ref, b2_ref, o_ref):
    h = jnp.maximum(
        jnp.dot(x_ref[...], w1_ref[...], preferred_element_type=jnp.float32)
        + b1_ref[...], 0.0)
    o_ref[...] = jnp.dot(h, w2_ref[...], preferred_element_type=jnp.float32) + b2_ref[...]


def _mlp(x, w1t, b1, w2t, b2):
    return pl.pallas_call(
        _mlp_body,
        out_shape=jax.ShapeDtypeStruct((VV, HH), jnp.float32),
    )(x, w1t, b1, w2t, b2)


MSG_TILE = 1024


def _msg_body(ef_ref, ns_ref, e1w_ref, e1b_ref, e2w_ref, e2b_ref, o_ref):
    ew = jnp.maximum(
        jnp.dot(ef_ref[...], e1w_ref[...], preferred_element_type=jnp.float32)
        + e1b_ref[...], 0.0)
    wm = jnp.dot(ew, e2w_ref[...], preferred_element_type=jnp.float32) + e2b_ref[...]
    ns = ns_ref[...]
    acc = ns[:, 0:1] * wm[:, 0:HH]
    for i in range(1, HH):
        acc = acc + ns[:, i:i + 1] * wm[:, i * HH:(i + 1) * HH]
    o_ref[...] = acc


def _msg(ef, nsrc, e1wt, e1b, e2wt, e2b):
    return pl.pallas_call(
        _msg_body,
        grid=(EE // MSG_TILE,),
        in_specs=[
            pl.BlockSpec((MSG_TILE, EIN), lambda i: (i, 0)),
            pl.BlockSpec((MSG_TILE, HH), lambda i: (i, 0)),
            pl.BlockSpec((EIN, EHID), lambda i: (0, 0)),
            pl.BlockSpec((1, EHID), lambda i: (0, 0)),
            pl.BlockSpec((EHID, HH * HH), lambda i: (0, 0)),
            pl.BlockSpec((1, HH * HH), lambda i: (0, 0)),
        ],
        out_specs=pl.BlockSpec((MSG_TILE, HH), lambda i: (i, 0)),
        out_shape=jax.ShapeDtypeStruct((EE, HH), jnp.float32),
    )(ef, nsrc, e1wt, e1b, e2wt, e2b)


def _gru_body(p0_ref, p1_ref, h_ref, cb_ref, wih_ref, whh_ref, bih_ref, bhh_ref, o_ref):
    m = jnp.maximum(p0_ref[...] + p1_ref[...] + cb_ref[...], 0.0)
    h = h_ref[...]
    gi = jnp.dot(m, wih_ref[...], preferred_element_type=jnp.float32) + bih_ref[...]
    gh = jnp.dot(h, whh_ref[...], preferred_element_type=jnp.float32) + bhh_ref[...]
    r = jax.nn.sigmoid(gi[:, 0:HH] + gh[:, 0:HH])
    z = jax.nn.sigmoid(gi[:, HH:2 * HH] + gh[:, HH:2 * HH])
    n = jnp.tanh(gi[:, 2 * HH:3 * HH] + r * gh[:, 2 * HH:3 * HH])
    o_ref[...] = (1.0 - z) * n + z * h


def _gru(p0, p1, h, cb, wiht, whht, bih, bhh):
    return pl.pallas_call(
        _gru_body,
        out_shape=jax.ShapeDtypeStruct((VV, HH), jnp.float32),
    )(p0, p1, h, cb, wiht, whht, bih, bhh)


DEC_CELLS = 8


def _dec_body(x_ref, u1_ref, u2_ref, u1b_ref, u2b4_ref, d1_ref, d1b_ref,
              d2_ref, d2b_ref, o_ref):
    x = x_ref[0]
    tt = jnp.dot(u1_ref[...], u2_ref[...], preferred_element_type=jnp.float32)
    b2v = jnp.dot(u1b_ref[...], u2_ref[...],
                  preferred_element_type=jnp.float32) + u2b4_ref[...]
    for p in range(4):
        cols = []
        for q in range(4):
            rblk = 2 * (p // 2) + (q // 2)
            cblk = 2 * (p % 2) + (q % 2)
            tpq = tt[rblk * HH:(rblk + 1) * HH, cblk * HH:(cblk + 1) * HH]
            y = jnp.dot(x, tpq, preferred_element_type=jnp.float32) \
                + b2v[:, cblk * HH:(cblk + 1) * HH]
            z = jnp.maximum(
                jnp.dot(y, d1_ref[...], preferred_element_type=jnp.float32)
                + d1b_ref[...], 0.0)
            cols.append(jnp.dot(z, d2_ref[...],
                                preferred_element_type=jnp.float32) + d2b_ref[...])
        o_ref[0, p] = jnp.concatenate(cols, axis=1)


def _decoder(h3, u1r, u2r, u1b, u2b4, d1wt, d1b, d2wt, d2b):
    nblk = h3.shape[0]
    rows = DEC_CELLS * HH
    return pl.pallas_call(
        _dec_body,
        grid=(nblk,),
        in_specs=[
            pl.BlockSpec((1, rows, HH), lambda i: (i, 0, 0)),
            pl.BlockSpec((4 * HH, HH), lambda i: (0, 0)),
            pl.BlockSpec((HH, 4 * HH), lambda i: (0, 0)),
            pl.BlockSpec((1, HH), lambda i: (0, 0)),
            pl.BlockSpec((1, 4 * HH), lambda i: (0, 0)),
            pl.BlockSpec((HH, 16), lambda i: (0, 0)),
            pl.BlockSpec((1, 16), lambda i: (0, 0)),
            pl.BlockSpec((16, NOUT), lambda i: (0, 0)),
            pl.BlockSpec((1, NOUT), lambda i: (0, 0)),
        ],
        out_specs=pl.BlockSpec((1, 4, rows, 4 * NOUT), lambda i: (i, 0, 0, 0)),
        out_shape=jax.ShapeDtypeStruct((nblk, 4, rows, 4 * NOUT), jnp.float32),
    )(h3, u1r, u2r, u1b, u2b4, d1wt, d1b, d2wt, d2b)


def _pool_matrix():
    pm = np.zeros((HH, 4 * RR), np.float32)
    for c4 in range(HH):
        for r in range(4):
            for dc in range(4):
                pm[c4, r * RR + c4 * 4 + dc] = 1.0 / 16.0
    return jnp.asarray(pm)


def _mpnn_block(node, edge_feats, src, dst3, zeros, w):
    (e1wt, e1b, e2wt, e2b, cb, wiht, whht, bih, bhh) = w
    for _ in range(2):
        nsrc = _sc_gather_call()(node, src)
        msg = _msg(edge_feats, nsrc, e1wt, e1b, e2wt, e2b)
        parts = _sc_scatter_call()(msg, dst3, zeros)
        node = _gru(parts[:VV], parts[VV:], node, cb, wiht, whht, bih, bhh)
    return node


def kernel(node_feats, edge_feats, edge_index,
           b0_p1W, b0_p1b, b0_p2W, b0_p2b, b0_e1W, b0_e1b, b0_e2W, b0_e2b,
           b0_cb, b0_Wih, b0_Whh, b0_bih, b0_bhh,
           b1_p1W, b1_p1b, b1_p2W, b1_p2b, b1_e1W, b1_e1b, b1_e2W, b1_e2b,
           b1_cb, b1_Wih, b1_Whh, b1_bih, b1_bhh,
           up1W, up1b, up2W, up2b, d1W, d1b, d2W, d2b):
    r2 = lambda v: v.reshape(1, -1)
    src = edge_index[0]
    dst3 = edge_index[1].reshape(NW, NCH, CH)
    zeros = jnp.zeros((VV, HH), jnp.float32)

    x3 = node_feats.reshape(6 * HH, 4 * RR, CC)
    h0 = _encoder(x3, _pool_matrix(), b0_p1W.T, r2(b0_p1b), b0_p2W.T,
                  r2(b0_p2b)).reshape(VV, HH)

    w0 = (b0_e1W.T, r2(b0_e1b), b0_e2W.T, r2(b0_e2b), r2(b0_cb),
          b0_Wih.T, b0_Whh.T, r2(b0_bih), r2(b0_bhh))
    node = _mpnn_block(h0, edge_feats, src, dst3, zeros, w0)

    h1 = _mlp(node, b1_p1W.T, r2(b1_p1b), b1_p2W.T, r2(b1_p2b))
    w1 = (b1_e1W.T, r2(b1_e1b), b1_e2W.T, r2(b1_e2b), r2(b1_cb),
          b1_Wih.T, b1_Whh.T, r2(b1_bih), r2(b1_bhh))
    node = _mpnn_block(h1, edge_feats, src, dst3, zeros, w1)

    u1r = up1W.transpose(2, 3, 0, 1).reshape(4 * HH, HH)
    u2r = up2W.transpose(0, 2, 3, 1).reshape(HH, 4 * HH)
    u2b4 = jnp.tile(up2b, (4,)).reshape(1, 4 * HH)
    h3 = node.reshape(192 // DEC_CELLS, DEC_CELLS * HH, HH)
    o5 = _decoder(h3, u1r, u2r, r2(up1b), u2b4, d1W.T, r2(d1b), d2W.T, r2(d2b))
    o = o5.reshape(24, 4, DEC_CELLS, HH, 4 * NOUT).transpose(0, 2, 1, 3, 4)
    return o.reshape(6 * RR * RR, NOUT)


# speedup vs baseline: 2.4282x; 2.4282x over previous
import functools

import jax
import jax.numpy as jnp
import numpy as np
from jax import lax
from jax.experimental import pallas as pl
from jax.experimental.pallas import tpu as pltpu
from jax.experimental.pallas import tpu_sc as plsc

RR = 128
CC = 128
HH = 32
EIN = 16
EHID = 64
NOUT = 3
VV = 6 * (RR // 4) ** 2
EE = VV * 16

NW = 32
EPW = EE // NW
CH = 128
NCH = EPW // CH
VS = VV // 16

@functools.cache
def _sc_gather_call():
    return pl.kernel(
        _sc_gather_body,
        out_type=jax.ShapeDtypeStruct((EE, HH), jnp.float32),
        mesh=plsc.VectorSubcoreMesh(core_axis_name="c", subcore_axis_name="s"),
        scratch_types=[
            pltpu.VMEM((EPW,), jnp.int32),
            pltpu.VMEM((EPW, HH), jnp.float32),
            pltpu.SemaphoreType.DMA,
        ],
        compiler_params=pltpu.CompilerParams(use_tc_tiling_on_sc=False),
    )


def _sc_gather_body(node_hbm, src_hbm, out_hbm, idx_v, rows_v, sem):
    wid = lax.axis_index("s") * 2 + lax.axis_index("c")
    base = wid * EPW
    pltpu.sync_copy(src_hbm.at[pl.ds(base, EPW)], idx_v)
    cps = []
    for j in range(NCH):
        cps.append(pltpu.async_copy(
            node_hbm.at[idx_v.at[pl.ds(j * CH, CH)]],
            rows_v.at[pl.ds(j * CH, CH)], sem))
    for cp in cps:
        cp.wait()
    pltpu.sync_copy(rows_v, out_hbm.at[pl.ds(base, EPW)])


@functools.cache
def _sc_scatter_call():
    return pl.kernel(
        _sc_scatter_body,
        out_type=jax.ShapeDtypeStruct((2 * VV, HH), jnp.float32),
        mesh=plsc.VectorSubcoreMesh(core_axis_name="c", subcore_axis_name="s"),
        scratch_types=[
            pltpu.VMEM((NCH, CH), jnp.int32),
            pltpu.VMEM((EPW, HH), jnp.float32),
            pltpu.VMEM_SHARED((VV, HH), jnp.float32),
        ],
        compiler_params=pltpu.CompilerParams(use_tc_tiling_on_sc=False),
    )


def _sc_scatter_body(msg_hbm, dst_hbm, zeros_hbm, out_hbm, idx_v, msg_v, acc):
    cid = lax.axis_index("c")
    sid = lax.axis_index("s")
    wid = sid * 2 + cid
    pltpu.sync_copy(zeros_hbm.at[pl.ds(sid * VS, VS)], acc.at[pl.ds(sid * VS, VS)])
    plsc.subcore_barrier()
    pltpu.sync_copy(dst_hbm.at[wid], idx_v)
    pltpu.sync_copy(msg_hbm.at[pl.ds(wid * EPW, EPW)], msg_v)
    for j in range(NCH):
        pltpu.sync_copy(msg_v.at[pl.ds(j * CH, CH)], acc.at[idx_v.at[j]], add=True)
    plsc.subcore_barrier()
    pltpu.sync_copy(acc.at[pl.ds(sid * VS, VS)],
                    out_hbm.at[pl.ds(cid * VV + sid * VS, VS)])


def _enc_body(x_ref, pm_ref, w1_ref, b1_ref, w2_ref, b2_ref, o_ref):
    x = x_ref[0]
    pooled = jnp.dot(pm_ref[...], x, preferred_element_type=jnp.float32)
    h = jnp.maximum(
        jnp.dot(pooled, w1_ref[...], preferred_element_type=jnp.float32)
        + b1_ref[...], 0.0)
    o_ref[0] = jnp.dot(h, w2_ref[...], preferred_element_type=jnp.float32) + b2_ref[...]


def _encoder(x3, pm, w1t, b1, w2t, b2):
    ncell = x3.shape[0]
    return pl.pallas_call(
        _enc_body,
        grid=(ncell,),
        in_specs=[
            pl.BlockSpec((1, 4 * RR, CC), lambda i: (i, 0, 0)),
            pl.BlockSpec((HH, 4 * RR), lambda i: (0, 0)),
            pl.BlockSpec((CC, HH), lambda i: (0, 0)),
            pl.BlockSpec((1, HH), lambda i: (0, 0)),
            pl.BlockSpec((HH, HH), lambda i: (0, 0)),
            pl.BlockSpec((1, HH), lambda i: (0, 0)),
        ],
        out_specs=pl.BlockSpec((1, HH, HH), lambda i: (i, 0, 0)),
        out_shape=jax.ShapeDtypeStruct((ncell, HH, HH), jnp.float32),
    )(x3, pm, w1t, b1, w2t, b2)


def _mlp_body(x_ref, w1_ref, b1_ref, w2_ref, b2_ref, o_ref):
    h = jnp.maximum(
        jnp.dot(x_ref[...], w1_ref[...], preferred_element_type=jnp.float32)
        + b1_ref[...], 0.0)
    o_ref[...] = jnp.dot(h, w2_ref[...], preferred_element_type=jnp.float32) + b2_ref[...]


def _mlp(x, w1t, b1, w2t, b2):
    return pl.pallas_call(
        _mlp_body,
        out_shape=jax.ShapeDtypeStruct((VV, HH), jnp.float32),
    )(x, w1t, b1, w2t, b2)


MSG_TILE = 1024


def _msg_body(ef_ref, ns_ref, e1w_ref, e1b_ref, e2w_ref, e2b_ref, kx_ref,
              ks_ref, o_ref):
    ew = jnp.maximum(
        jnp.dot(ef_ref[...], e1w_ref[...], preferred_element_type=jnp.float32)
        + e1b_ref[...], 0.0)
    wm = jnp.dot(ew, e2w_ref[...], preferred_element_type=jnp.float32) + e2b_ref[...]
    x = jnp.dot(ns_ref[...], kx_ref[...], preferred_element_type=jnp.float32)
    o_ref[...] = jnp.dot(x * wm, ks_ref[...], preferred_element_type=jnp.float32)


def _msg(ef, nsrc, e1wt, e1b, e2wt, e2b, kx, ks):
    return pl.pallas_call(
        _msg_body,
        grid=(EE // MSG_TILE,),
        in_specs=[
            pl.BlockSpec((MSG_TILE, EIN), lambda i: (i, 0)),
            pl.BlockSpec((MSG_TILE, HH), lambda i: (i, 0)),
            pl.BlockSpec((EIN, EHID), lambda i: (0, 0)),
            pl.BlockSpec((1, EHID), lambda i: (0, 0)),
            pl.BlockSpec((EHID, HH * HH), lambda i: (0, 0)),
            pl.BlockSpec((1, HH * HH), lambda i: (0, 0)),
            pl.BlockSpec((HH, HH * HH), lambda i: (0, 0)),
            pl.BlockSpec((HH * HH, HH), lambda i: (0, 0)),
        ],
        out_specs=pl.BlockSpec((MSG_TILE, HH), lambda i: (i, 0)),
        out_shape=jax.ShapeDtypeStruct((EE, HH), jnp.float32),
    )(ef, nsrc, e1wt, e1b, e2wt, e2b, kx, ks)


def _gru_body(p0_ref, p1_ref, h_ref, cb_ref, wih_ref, whh_ref, bih_ref, bhh_ref, o_ref):
    m = jnp.maximum(p0_ref[...] + p1_ref[...] + cb_ref[...], 0.0)
    h = h_ref[...]
    gi = jnp.dot(m, wih_ref[...], preferred_element_type=jnp.float32) + bih_ref[...]
    gh = jnp.dot(h, whh_ref[...], preferred_element_type=jnp.float32) + bhh_ref[...]
    r = jax.nn.sigmoid(gi[:, 0:HH] + gh[:, 0:HH])
    z = jax.nn.sigmoid(gi[:, HH:2 * HH] + gh[:, HH:2 * HH])
    n = jnp.tanh(gi[:, 2 * HH:3 * HH] + r * gh[:, 2 * HH:3 * HH])
    o_ref[...] = (1.0 - z) * n + z * h


def _gru(p0, p1, h, cb, wiht, whht, bih, bhh):
    return pl.pallas_call(
        _gru_body,
        out_shape=jax.ShapeDtypeStruct((VV, HH), jnp.float32),
    )(p0, p1, h, cb, wiht, whht, bih, bhh)


DEC_CELLS = 8


def _dec_body(x_ref, u1_ref, u2_ref, u1b_ref, u2b4_ref, d1_ref, d1b_ref,
              d2_ref, d2b_ref, o_ref):
    x = x_ref[0]
    tt = jnp.dot(u1_ref[...], u2_ref[...], preferred_element_type=jnp.float32)
    b2v = jnp.dot(u1b_ref[...], u2_ref[...],
                  preferred_element_type=jnp.float32) + u2b4_ref[...]
    for p in range(4):
        cols = []
        for q in range(4):
            rblk = 2 * (p // 2) + (q // 2)
            cblk = 2 * (p % 2) + (q % 2)
            tpq = tt[rblk * HH:(rblk + 1) * HH, cblk * HH:(cblk + 1) * HH]
            y = jnp.dot(x, tpq, preferred_element_type=jnp.float32) \
                + b2v[:, cblk * HH:(cblk + 1) * HH]
            z = jnp.maximum(
                jnp.dot(y, d1_ref[...], preferred_element_type=jnp.float32)
                + d1b_ref[...], 0.0)
            cols.append(jnp.dot(z, d2_ref[...],
                                preferred_element_type=jnp.float32) + d2b_ref[...])
        o_ref[0, p] = jnp.concatenate(cols, axis=1)


def _decoder(h3, u1r, u2r, u1b, u2b4, d1wt, d1b, d2wt, d2b):
    nblk = h3.shape[0]
    rows = DEC_CELLS * HH
    return pl.pallas_call(
        _dec_body,
        grid=(nblk,),
        in_specs=[
            pl.BlockSpec((1, rows, HH), lambda i: (i, 0, 0)),
            pl.BlockSpec((4 * HH, HH), lambda i: (0, 0)),
            pl.BlockSpec((HH, 4 * HH), lambda i: (0, 0)),
            pl.BlockSpec((1, HH), lambda i: (0, 0)),
            pl.BlockSpec((1, 4 * HH), lambda i: (0, 0)),
            pl.BlockSpec((HH, 16), lambda i: (0, 0)),
            pl.BlockSpec((1, 16), lambda i: (0, 0)),
            pl.BlockSpec((16, NOUT), lambda i: (0, 0)),
            pl.BlockSpec((1, NOUT), lambda i: (0, 0)),
        ],
        out_specs=pl.BlockSpec((1, 4, rows, 4 * NOUT), lambda i: (i, 0, 0, 0)),
        out_shape=jax.ShapeDtypeStruct((nblk, 4, rows, 4 * NOUT), jnp.float32),
    )(h3, u1r, u2r, u1b, u2b4, d1wt, d1b, d2wt, d2b)


def _pool_matrix():
    pm = np.zeros((HH, 4 * RR), np.float32)
    for c4 in range(HH):
        for r in range(4):
            for dc in range(4):
                pm[c4, r * RR + c4 * 4 + dc] = 1.0 / 16.0
    return jnp.asarray(pm)


def _mpnn_block(node, edge_feats, src, dst3, zeros, kx, ks, w):
    (e1wt, e1b, e2wt, e2b, cb, wiht, whht, bih, bhh) = w
    for _ in range(2):
        nsrc = _sc_gather_call()(node, src)
        msg = _msg(edge_feats, nsrc, e1wt, e1b, e2wt, e2b, kx, ks)
        parts = _sc_scatter_call()(msg, dst3, zeros)
        node = _gru(parts[:VV], parts[VV:], node, cb, wiht, whht, bih, bhh)
    return node


def kernel(node_feats, edge_feats, edge_index,
           b0_p1W, b0_p1b, b0_p2W, b0_p2b, b0_e1W, b0_e1b, b0_e2W, b0_e2b,
           b0_cb, b0_Wih, b0_Whh, b0_bih, b0_bhh,
           b1_p1W, b1_p1b, b1_p2W, b1_p2b, b1_e1W, b1_e1b, b1_e2W, b1_e2b,
           b1_cb, b1_Wih, b1_Whh, b1_bih, b1_bhh,
           up1W, up1b, up2W, up2b, d1W, d1b, d2W, d2b):
    r2 = lambda v: v.reshape(1, -1)
    src = edge_index[0]
    dst3 = edge_index[1].reshape(NW, NCH, CH)
    zeros = jnp.zeros((VV, HH), jnp.float32)
    kx = jnp.kron(jnp.eye(HH, dtype=jnp.float32), jnp.ones((1, HH), jnp.float32))
    ks = jnp.kron(jnp.ones((HH, 1), jnp.float32), jnp.eye(HH, dtype=jnp.float32))

    x3 = node_feats.reshape(6 * HH, 4 * RR, CC)
    h0 = _encoder(x3, _pool_matrix(), b0_p1W.T, r2(b0_p1b), b0_p2W.T,
                  r2(b0_p2b)).reshape(VV, HH)

    w0 = (b0_e1W.T, r2(b0_e1b), b0_e2W.T, r2(b0_e2b), r2(b0_cb),
          b0_Wih.T, b0_Whh.T, r2(b0_bih), r2(b0_bhh))
    node = _mpnn_block(h0, edge_feats, src, dst3, zeros, kx, ks, w0)

    h1 = _mlp(node, b1_p1W.T, r2(b1_p1b), b1_p2W.T, r2(b1_p2b))
    w1 = (b1_e1W.T, r2(b1_e1b), b1_e2W.T, r2(b1_e2b), r2(b1_cb),
          b1_Wih.T, b1_Whh.T, r2(b1_bih), r2(b1_bhh))
    node = _mpnn_block(h1, edge_feats, src, dst3, zeros, kx, ks, w1)

    u1r = up1W.transpose(2, 3, 0, 1).reshape(4 * HH, HH)
    u2r = up2W.transpose(0, 2, 3, 1).reshape(HH, 4 * HH)
    u2b4 = jnp.tile(up2b, (4,)).reshape(1, 4 * HH)
    h3 = node.reshape(192 // DEC_CELLS, DEC_CELLS * HH, HH)
    o5 = _decoder(h3, u1r, u2r, r2(up1b), u2b4, d1W.T, r2(d1b), d2W.T, r2(d2b))
    o = o5.reshape(24, 4, DEC_CELLS, HH, 4 * NOUT).transpose(0, 2, 1, 3, 4)
    return o.reshape(6 * RR * RR, NOUT)


# speedup vs baseline: 2.6329x; 1.0843x over previous
import functools

import jax
import jax.numpy as jnp
import numpy as np
from jax import lax
from jax.experimental import pallas as pl
from jax.experimental.pallas import tpu as pltpu
from jax.experimental.pallas import tpu_sc as plsc

RR = 128
CC = 128
HH = 32
EIN = 16
EHID = 64
NOUT = 3
VV = 6 * (RR // 4) ** 2
EE = VV * 16

NW = 32
EPW = EE // NW
CH = 128
NCH = EPW // CH
VS = VV // 16

@functools.cache
def _sc_gather_call():
    return pl.kernel(
        _sc_gather_body,
        out_type=jax.ShapeDtypeStruct((EE, HH), jnp.float32),
        mesh=plsc.VectorSubcoreMesh(core_axis_name="c", subcore_axis_name="s"),
        scratch_types=[
            pltpu.VMEM((EPW,), jnp.int32),
            pltpu.VMEM((EPW, HH), jnp.float32),
            pltpu.SemaphoreType.DMA,
        ],
        compiler_params=pltpu.CompilerParams(use_tc_tiling_on_sc=False),
    )


def _sc_gather_body(node_hbm, src_hbm, out_hbm, idx_v, rows_v, sem):
    wid = lax.axis_index("s") * 2 + lax.axis_index("c")
    base = wid * EPW
    pltpu.sync_copy(src_hbm.at[pl.ds(base, EPW)], idx_v)
    cps = []
    for j in range(NCH):
        cps.append(pltpu.async_copy(
            node_hbm.at[idx_v.at[pl.ds(j * CH, CH)]],
            rows_v.at[pl.ds(j * CH, CH)], sem))
    for cp in cps:
        cp.wait()
    pltpu.sync_copy(rows_v, out_hbm.at[pl.ds(base, EPW)])


@functools.cache
def _sc_scatter_call():
    return pl.kernel(
        _sc_scatter_body,
        out_type=jax.ShapeDtypeStruct((2 * VV, HH), jnp.float32),
        mesh=plsc.VectorSubcoreMesh(core_axis_name="c", subcore_axis_name="s"),
        scratch_types=[
            pltpu.VMEM((NCH, CH), jnp.int32),
            pltpu.VMEM((EPW, HH), jnp.float32),
            pltpu.VMEM_SHARED((VV, HH), jnp.float32),
        ],
        compiler_params=pltpu.CompilerParams(use_tc_tiling_on_sc=False),
    )


def _sc_scatter_body(msg_hbm, dst_hbm, zeros_hbm, out_hbm, idx_v, msg_v, acc):
    cid = lax.axis_index("c")
    sid = lax.axis_index("s")
    wid = sid * 2 + cid
    pltpu.sync_copy(zeros_hbm.at[pl.ds(sid * VS, VS)], acc.at[pl.ds(sid * VS, VS)])
    plsc.subcore_barrier()
    pltpu.sync_copy(dst_hbm.at[wid], idx_v)
    pltpu.sync_copy(msg_hbm.at[pl.ds(wid * EPW, EPW)], msg_v)
    for j in range(NCH):
        pltpu.sync_copy(msg_v.at[pl.ds(j * CH, CH)], acc.at[idx_v.at[j]], add=True)
    plsc.subcore_barrier()
    pltpu.sync_copy(acc.at[pl.ds(sid * VS, VS)],
                    out_hbm.at[pl.ds(cid * VV + sid * VS, VS)])


def _enc_body(x_ref, pm_ref, w1_ref, b1_ref, w2_ref, b2_ref, o_ref):
    x = x_ref[0]
    pooled = jnp.dot(pm_ref[...], x, preferred_element_type=jnp.float32)
    h = jnp.maximum(
        jnp.dot(pooled, w1_ref[...], preferred_element_type=jnp.float32)
        + b1_ref[...], 0.0)
    o_ref[0] = jnp.dot(h, w2_ref[...], preferred_element_type=jnp.float32) + b2_ref[...]


def _encoder(x3, pm, w1t, b1, w2t, b2):
    ncell = x3.shape[0]
    return pl.pallas_call(
        _enc_body,
        grid=(ncell,),
        in_specs=[
            pl.BlockSpec((1, 4 * RR, CC), lambda i: (i, 0, 0)),
            pl.BlockSpec((HH, 4 * RR), lambda i: (0, 0)),
            pl.BlockSpec((CC, HH), lambda i: (0, 0)),
            pl.BlockSpec((1, HH), lambda i: (0, 0)),
            pl.BlockSpec((HH, HH), lambda i: (0, 0)),
            pl.BlockSpec((1, HH), lambda i: (0, 0)),
        ],
        out_specs=pl.BlockSpec((1, HH, HH), lambda i: (i, 0, 0)),
        out_shape=jax.ShapeDtypeStruct((ncell, HH, HH), jnp.float32),
    )(x3, pm, w1t, b1, w2t, b2)


def _mlp_body(x_ref, w1_ref, b1_ref, w2_ref, b2_ref, o_ref):
    h = jnp.maximum(
        jnp.dot(x_ref[...], w1_ref[...], preferred_element_type=jnp.float32)
        + b1_ref[...], 0.0)
    o_ref[...] = jnp.dot(h, w2_ref[...], preferred_element_type=jnp.float32) + b2_ref[...]


def _mlp(x, w1t, b1, w2t, b2):
    return pl.pallas_call(
        _mlp_body,
        out_shape=jax.ShapeDtypeStruct((VV, HH), jnp.float32),
    )(x, w1t, b1, w2t, b2)


MSG_TILE = 2048


def _msg_body(ef_ref, ns_ref, e1w_ref, e1b_ref, e2w_ref, e2br_ref, kx_ref,
              ks_ref, o_ref):
    ew = jnp.maximum(
        jnp.dot(ef_ref[...], e1w_ref[...], preferred_element_type=jnp.float32)
        + e1b_ref[...], 0.0)
    wm = jnp.dot(ew.astype(jnp.bfloat16), e2w_ref[...],
                 preferred_element_type=jnp.float32)
    ns = ns_ref[...].astype(jnp.bfloat16)
    x = jnp.dot(ns, kx_ref[...], preferred_element_type=jnp.float32)
    o_ref[...] = (
        jnp.dot((x * wm).astype(jnp.bfloat16), ks_ref[...],
                preferred_element_type=jnp.float32)
        + jnp.dot(ns, e2br_ref[...], preferred_element_type=jnp.float32))


def _msg(ef, nsrc, e1wt, e1b, e2wt, e2b, kx, ks):
    return pl.pallas_call(
        _msg_body,
        grid=(EE // MSG_TILE,),
        in_specs=[
            pl.BlockSpec((MSG_TILE, EIN), lambda i: (i, 0)),
            pl.BlockSpec((MSG_TILE, HH), lambda i: (i, 0)),
            pl.BlockSpec((EIN, EHID), lambda i: (0, 0)),
            pl.BlockSpec((1, EHID), lambda i: (0, 0)),
            pl.BlockSpec((EHID, HH * HH), lambda i: (0, 0)),
            pl.BlockSpec((HH, HH), lambda i: (0, 0)),
            pl.BlockSpec((HH, HH * HH), lambda i: (0, 0)),
            pl.BlockSpec((HH * HH, HH), lambda i: (0, 0)),
        ],
        out_specs=pl.BlockSpec((MSG_TILE, HH), lambda i: (i, 0)),
        out_shape=jax.ShapeDtypeStruct((EE, HH), jnp.float32),
    )(ef, nsrc, e1wt, e1b, e2wt, e2b, kx, ks)


def _gru_body(p0_ref, p1_ref, h_ref, cb_ref, wih_ref, whh_ref, bih_ref, bhh_ref, o_ref):
    m = jnp.maximum(p0_ref[...] + p1_ref[...] + cb_ref[...], 0.0)
    h = h_ref[...]
    gi = jnp.dot(m, wih_ref[...], preferred_element_type=jnp.float32) + bih_ref[...]
    gh = jnp.dot(h, whh_ref[...], preferred_element_type=jnp.float32) + bhh_ref[...]
    r = jax.nn.sigmoid(gi[:, 0:HH] + gh[:, 0:HH])
    z = jax.nn.sigmoid(gi[:, HH:2 * HH] + gh[:, HH:2 * HH])
    n = jnp.tanh(gi[:, 2 * HH:3 * HH] + r * gh[:, 2 * HH:3 * HH])
    o_ref[...] = (1.0 - z) * n + z * h


def _gru(p0, p1, h, cb, wiht, whht, bih, bhh):
    return pl.pallas_call(
        _gru_body,
        out_shape=jax.ShapeDtypeStruct((VV, HH), jnp.float32),
    )(p0, p1, h, cb, wiht, whht, bih, bhh)


DEC_CELLS = 8


def _dec_body(x_ref, u1_ref, u2_ref, u1b_ref, u2b4_ref, d1_ref, d1b_ref,
              d2_ref, d2b_ref, o_ref):
    x = x_ref[0]
    tt = jnp.dot(u1_ref[...], u2_ref[...], preferred_element_type=jnp.float32)
    b2v = jnp.dot(u1b_ref[...], u2_ref[...],
                  preferred_element_type=jnp.float32) + u2b4_ref[...]
    for p in range(4):
        cols = []
        for q in range(4):
            rblk = 2 * (p // 2) + (q // 2)
            cblk = 2 * (p % 2) + (q % 2)
            tpq = tt[rblk * HH:(rblk + 1) * HH, cblk * HH:(cblk + 1) * HH]
            y = jnp.dot(x, tpq, preferred_element_type=jnp.float32) \
                + b2v[:, cblk * HH:(cblk + 1) * HH]
            z = jnp.maximum(
                jnp.dot(y, d1_ref[...], preferred_element_type=jnp.float32)
                + d1b_ref[...], 0.0)
            cols.append(jnp.dot(z, d2_ref[...],
                                preferred_element_type=jnp.float32) + d2b_ref[...])
        o_ref[0, p] = jnp.concatenate(cols, axis=1)


def _decoder(h3, u1r, u2r, u1b, u2b4, d1wt, d1b, d2wt, d2b):
    nblk = h3.shape[0]
    rows = DEC_CELLS * HH
    return pl.pallas_call(
        _dec_body,
        grid=(nblk,),
        in_specs=[
            pl.BlockSpec((1, rows, HH), lambda i: (i, 0, 0)),
            pl.BlockSpec((4 * HH, HH), lambda i: (0, 0)),
            pl.BlockSpec((HH, 4 * HH), lambda i: (0, 0)),
            pl.BlockSpec((1, HH), lambda i: (0, 0)),
            pl.BlockSpec((1, 4 * HH), lambda i: (0, 0)),
            pl.BlockSpec((HH, 16), lambda i: (0, 0)),
            pl.BlockSpec((1, 16), lambda i: (0, 0)),
            pl.BlockSpec((16, NOUT), lambda i: (0, 0)),
            pl.BlockSpec((1, NOUT), lambda i: (0, 0)),
        ],
        out_specs=pl.BlockSpec((1, 4, rows, 4 * NOUT), lambda i: (i, 0, 0, 0)),
        out_shape=jax.ShapeDtypeStruct((nblk, 4, rows, 4 * NOUT), jnp.float32),
    )(h3, u1r, u2r, u1b, u2b4, d1wt, d1b, d2wt, d2b)


def _pool_matrix():
    pm = np.zeros((HH, 4 * RR), np.float32)
    for c4 in range(HH):
        for r in range(4):
            for dc in range(4):
                pm[c4, r * RR + c4 * 4 + dc] = 1.0 / 16.0
    return jnp.asarray(pm)


def _mpnn_block(node, edge_feats, src, dst3, zeros, kx, ks, w):
    (e1wt, e1b, e2wt, e2b, cb, wiht, whht, bih, bhh) = w
    e2wt = e2wt.astype(jnp.bfloat16)
    e2b = e2b.reshape(HH, HH).astype(jnp.bfloat16)
    for _ in range(2):
        nsrc = _sc_gather_call()(node, src)
        msg = _msg(edge_feats, nsrc, e1wt, e1b, e2wt, e2b, kx, ks)
        parts = _sc_scatter_call()(msg, dst3, zeros)
        node = _gru(parts[:VV], parts[VV:], node, cb, wiht, whht, bih, bhh)
    return node


def kernel(node_feats, edge_feats, edge_index,
           b0_p1W, b0_p1b, b0_p2W, b0_p2b, b0_e1W, b0_e1b, b0_e2W, b0_e2b,
           b0_cb, b0_Wih, b0_Whh, b0_bih, b0_bhh,
           b1_p1W, b1_p1b, b1_p2W, b1_p2b, b1_e1W, b1_e1b, b1_e2W, b1_e2b,
           b1_cb, b1_Wih, b1_Whh, b1_bih, b1_bhh,
           up1W, up1b, up2W, up2b, d1W, d1b, d2W, d2b):
    r2 = lambda v: v.reshape(1, -1)
    src = edge_index[0]
    dst3 = edge_index[1].reshape(NW, NCH, CH)
    zeros = jnp.zeros((VV, HH), jnp.float32)
    kx = jnp.kron(jnp.eye(HH, dtype=jnp.bfloat16), jnp.ones((1, HH), jnp.bfloat16))
    ks = jnp.kron(jnp.ones((HH, 1), jnp.bfloat16), jnp.eye(HH, dtype=jnp.bfloat16))

    x3 = node_feats.reshape(6 * HH, 4 * RR, CC)
    h0 = _encoder(x3, _pool_matrix(), b0_p1W.T, r2(b0_p1b), b0_p2W.T,
                  r2(b0_p2b)).reshape(VV, HH)

    w0 = (b0_e1W.T, r2(b0_e1b), b0_e2W.T, r2(b0_e2b), r2(b0_cb),
          b0_Wih.T, b0_Whh.T, r2(b0_bih), r2(b0_bhh))
    node = _mpnn_block(h0, edge_feats, src, dst3, zeros, kx, ks, w0)

    h1 = _mlp(node, b1_p1W.T, r2(b1_p1b), b1_p2W.T, r2(b1_p2b))
    w1 = (b1_e1W.T, r2(b1_e1b), b1_e2W.T, r2(b1_e2b), r2(b1_cb),
          b1_Wih.T, b1_Whh.T, r2(b1_bih), r2(b1_bhh))
    node = _mpnn_block(h1, edge_feats, src, dst3, zeros, kx, ks, w1)

    u1r = up1W.transpose(2, 3, 0, 1).reshape(4 * HH, HH)
    u2r = up2W.transpose(0, 2, 3, 1).reshape(HH, 4 * HH)
    u2b4 = jnp.tile(up2b, (4,)).reshape(1, 4 * HH)
    h3 = node.reshape(192 // DEC_CELLS, DEC_CELLS * HH, HH)
    o5 = _decoder(h3, u1r, u2r, r2(up1b), u2b4, d1W.T, r2(d1b), d2W.T, r2(d2b))
    o = o5.reshape(24, 4, DEC_CELLS, HH, 4 * NOUT).transpose(0, 2, 1, 3, 4)
    return o.reshape(6 * RR * RR, NOUT)


# speedup vs baseline: 3.2180x; 1.2223x over previous
import functools

import jax
import jax.numpy as jnp
import numpy as np
from jax import lax
from jax.experimental import pallas as pl
from jax.experimental.pallas import tpu as pltpu
from jax.experimental.pallas import tpu_sc as plsc

RR = 128
CC = 128
HH = 32
EIN = 16
EHID = 64
NOUT = 3
VV = 6 * (RR // 4) ** 2
EE = VV * 16

NW = 32
EPW = EE // NW
CH = 128
NCH = EPW // CH
VS = VV // 16

@functools.cache
def _sc_gather_call():
    return pl.kernel(
        _sc_gather_body,
        out_type=jax.ShapeDtypeStruct((EE, HH), jnp.float32),
        mesh=plsc.VectorSubcoreMesh(core_axis_name="c", subcore_axis_name="s"),
        scratch_types=[
            pltpu.VMEM((EPW,), jnp.int32),
            pltpu.VMEM((EPW, HH), jnp.float32),
            pltpu.SemaphoreType.DMA,
        ],
        compiler_params=pltpu.CompilerParams(use_tc_tiling_on_sc=False),
    )


def _sc_gather_body(node_hbm, src_hbm, out_hbm, idx_v, rows_v, sem):
    wid = lax.axis_index("s") * 2 + lax.axis_index("c")
    base = wid * EPW
    pltpu.sync_copy(src_hbm.at[pl.ds(base, EPW)], idx_v)
    cps = []
    for j in range(NCH):
        cps.append(pltpu.async_copy(
            node_hbm.at[idx_v.at[pl.ds(j * CH, CH)]],
            rows_v.at[pl.ds(j * CH, CH)], sem))
    for cp in cps:
        cp.wait()
    pltpu.sync_copy(rows_v, out_hbm.at[pl.ds(base, EPW)])


@functools.cache
def _sc_scatter_call():
    return pl.kernel(
        _sc_scatter_body,
        out_type=jax.ShapeDtypeStruct((2 * VV, HH), jnp.float32),
        mesh=plsc.VectorSubcoreMesh(core_axis_name="c", subcore_axis_name="s"),
        scratch_types=[
            pltpu.VMEM((NCH, CH), jnp.int32),
            pltpu.VMEM((EPW, HH), jnp.float32),
            pltpu.VMEM_SHARED((VV, HH), jnp.float32),
        ],
        compiler_params=pltpu.CompilerParams(use_tc_tiling_on_sc=False),
    )


def _sc_scatter_body(msg_hbm, dst_hbm, zeros_hbm, out_hbm, idx_v, msg_v, acc):
    cid = lax.axis_index("c")
    sid = lax.axis_index("s")
    wid = sid * 2 + cid
    pltpu.sync_copy(zeros_hbm.at[pl.ds(sid * VS, VS)], acc.at[pl.ds(sid * VS, VS)])
    plsc.subcore_barrier()
    pltpu.sync_copy(dst_hbm.at[wid], idx_v)
    pltpu.sync_copy(msg_hbm.at[pl.ds(wid * EPW, EPW)], msg_v)
    for j in range(NCH):
        pltpu.sync_copy(msg_v.at[pl.ds(j * CH, CH)], acc.at[idx_v.at[j]], add=True)
    plsc.subcore_barrier()
    pltpu.sync_copy(acc.at[pl.ds(sid * VS, VS)],
                    out_hbm.at[pl.ds(cid * VV + sid * VS, VS)])


ENC_CELLS = 8


def _pack4(h):
    h3 = h.reshape(h.shape[0] // 4, 4, HH)
    return jnp.concatenate([h3[:, j, :] for j in range(4)], axis=1)


def _enc_body(x_ref, pm_ref, w1_ref, b1_ref, w2_ref, b2_ref, op_ref):
    for c in range(ENC_CELLS):
        x = x_ref[c]
        pooled = jnp.dot(pm_ref[...], x, preferred_element_type=jnp.float32)
        h = jnp.maximum(
            jnp.dot(pooled, w1_ref[...], preferred_element_type=jnp.float32)
            + b1_ref[...], 0.0)
        h = jnp.dot(h, w2_ref[...], preferred_element_type=jnp.float32) + b2_ref[...]
        op_ref[c * (HH // 4):(c + 1) * (HH // 4), :] = _pack4(h)


def _encoder(x3, pm, w1t, b1, w2t, b2):
    nblk = x3.shape[0] // ENC_CELLS
    return pl.pallas_call(
        _enc_body,
        grid=(nblk,),
        in_specs=[
            pl.BlockSpec((ENC_CELLS, 4 * RR, CC), lambda i: (i, 0, 0)),
            pl.BlockSpec((HH, 4 * RR), lambda i: (0, 0)),
            pl.BlockSpec((CC, HH), lambda i: (0, 0)),
            pl.BlockSpec((1, HH), lambda i: (0, 0)),
            pl.BlockSpec((HH, HH), lambda i: (0, 0)),
            pl.BlockSpec((1, HH), lambda i: (0, 0)),
        ],
        out_specs=pl.BlockSpec((ENC_CELLS * HH // 4, 4 * HH), lambda i: (i, 0)),
        out_shape=jax.ShapeDtypeStruct((VV // 4, 4 * HH), jnp.float32),
    )(x3, pm, w1t, b1, w2t, b2)


def _mlp_body(x_ref, w1_ref, b1_ref, w2_ref, b2_ref, op_ref):
    h = jnp.maximum(
        jnp.dot(x_ref[...], w1_ref[...], preferred_element_type=jnp.float32)
        + b1_ref[...], 0.0)
    op_ref[...] = jnp.dot(h, w2_ref[...], preferred_element_type=jnp.float32) \
        + b2_ref[...]


def _mlp(xp, w1t4, b14, w2t4, b24):
    return pl.pallas_call(
        _mlp_body,
        out_shape=jax.ShapeDtypeStruct((VV // 4, 4 * HH), jnp.float32),
    )(xp, w1t4, b14, w2t4, b24)


MSG_TILE = 2048


def _msg_body(ef_ref, ns_ref, e1w_ref, e1b_ref, e2w_ref, e2br_ref, kx_ref,
              ks_ref, o_ref):
    ew = jnp.maximum(
        jnp.dot(ef_ref[...], e1w_ref[...], preferred_element_type=jnp.float32)
        + e1b_ref[...], 0.0)
    wm = jnp.dot(ew.astype(jnp.bfloat16), e2w_ref[...],
                 preferred_element_type=jnp.float32)
    ns128 = ns_ref[...]
    ns = jnp.stack([ns128[:, j * HH:(j + 1) * HH] for j in range(4)],
                   axis=1).reshape(MSG_TILE, HH).astype(jnp.bfloat16)
    x = jnp.dot(ns, kx_ref[...], preferred_element_type=jnp.float32)
    msg = (
        jnp.dot((x * wm).astype(jnp.bfloat16), ks_ref[...],
                preferred_element_type=jnp.float32)
        + jnp.dot(ns, e2br_ref[...], preferred_element_type=jnp.float32))
    m3 = msg.reshape(MSG_TILE // 4, 4, HH)
    o_ref[...] = jnp.concatenate([m3[:, j, :] for j in range(4)], axis=1)


def _msg(ef, nsrc, e1wt, e1b, e2wt, e2b, kx, ks):
    return pl.pallas_call(
        _msg_body,
        grid=(EE // MSG_TILE,),
        in_specs=[
            pl.BlockSpec((MSG_TILE, EIN), lambda i: (i, 0)),
            pl.BlockSpec((MSG_TILE // 4, 4 * HH), lambda i: (i, 0)),
            pl.BlockSpec((EIN, EHID), lambda i: (0, 0)),
            pl.BlockSpec((1, EHID), lambda i: (0, 0)),
            pl.BlockSpec((EHID, HH * HH), lambda i: (0, 0)),
            pl.BlockSpec((HH, HH), lambda i: (0, 0)),
            pl.BlockSpec((HH, HH * HH), lambda i: (0, 0)),
            pl.BlockSpec((HH * HH, HH), lambda i: (0, 0)),
        ],
        out_specs=pl.BlockSpec((MSG_TILE // 4, 4 * HH), lambda i: (i, 0)),
        out_shape=jax.ShapeDtypeStruct((EE // 4, 4 * HH), jnp.float32),
    )(ef, nsrc, e1wt, e1b, e2wt, e2b, kx, ks)


def _gru_body(pp_ref, h_ref, cb4_ref, wih4_ref, whh4_ref, bih4_ref, bhh4_ref,
              op_ref):
    pp = pp_ref[...]
    m = jnp.maximum(pp[:VV // 4] + pp[VV // 4:] + cb4_ref[...], 0.0)
    h = h_ref[...]
    gi = jnp.dot(m, wih4_ref[...], preferred_element_type=jnp.float32) \
        + bih4_ref[...]
    gh = jnp.dot(h, whh4_ref[...], preferred_element_type=jnp.float32) \
        + bhh4_ref[...]
    for j in range(4):
        gij = gi[:, 3 * HH * j:3 * HH * (j + 1)]
        ghj = gh[:, 3 * HH * j:3 * HH * (j + 1)]
        r = jax.nn.sigmoid(gij[:, 0:HH] + ghj[:, 0:HH])
        z = jax.nn.sigmoid(gij[:, HH:2 * HH] + ghj[:, HH:2 * HH])
        n = jnp.tanh(gij[:, 2 * HH:3 * HH] + r * ghj[:, 2 * HH:3 * HH])
        hj = h[:, HH * j:HH * (j + 1)]
        op_ref[:, HH * j:HH * (j + 1)] = (1.0 - z) * n + z * hj


def _gru(pparts, hp, cb4, wih4, whh4, bih4, bhh4):
    return pl.pallas_call(
        _gru_body,
        out_shape=jax.ShapeDtypeStruct((VV // 4, 4 * HH), jnp.float32),
    )(pparts, hp, cb4, wih4, whh4, bih4, bhh4)


DEC_CELLS = 8


def _dec_body(x_ref, u1_ref, u2_ref, u1b_ref, u2b4_ref, d1_ref, d1b_ref,
              d2_ref, d2b_ref, o_ref):
    xp = x_ref[...]
    x = jnp.stack([xp[:, j * HH:(j + 1) * HH] for j in range(4)],
                  axis=1).reshape(DEC_CELLS * HH, HH)
    tt = jnp.dot(u1_ref[...], u2_ref[...], preferred_element_type=jnp.float32)
    b2v = jnp.dot(u1b_ref[...], u2_ref[...],
                  preferred_element_type=jnp.float32) + u2b4_ref[...]
    for p in range(4):
        cols = []
        for q in range(4):
            rblk = 2 * (p // 2) + (q // 2)
            cblk = 2 * (p % 2) + (q % 2)
            tpq = tt[rblk * HH:(rblk + 1) * HH, cblk * HH:(cblk + 1) * HH]
            y = jnp.dot(x, tpq, preferred_element_type=jnp.float32) \
                + b2v[:, cblk * HH:(cblk + 1) * HH]
            z = jnp.maximum(
                jnp.dot(y, d1_ref[...], preferred_element_type=jnp.float32)
                + d1b_ref[...], 0.0)
            cols.append(jnp.dot(z, d2_ref[...],
                                preferred_element_type=jnp.float32) + d2b_ref[...])
        o_ref[0, p] = jnp.concatenate(cols, axis=1)


def _decoder(hp, u1r, u2r, u1b, u2b4, d1wt, d1b, d2wt, d2b):
    rows = DEC_CELLS * HH
    nblk = hp.shape[0] // (rows // 4)
    return pl.pallas_call(
        _dec_body,
        grid=(nblk,),
        in_specs=[
            pl.BlockSpec((rows // 4, 4 * HH), lambda i: (i, 0)),
            pl.BlockSpec((4 * HH, HH), lambda i: (0, 0)),
            pl.BlockSpec((HH, 4 * HH), lambda i: (0, 0)),
            pl.BlockSpec((1, HH), lambda i: (0, 0)),
            pl.BlockSpec((1, 4 * HH), lambda i: (0, 0)),
            pl.BlockSpec((HH, 16), lambda i: (0, 0)),
            pl.BlockSpec((1, 16), lambda i: (0, 0)),
            pl.BlockSpec((16, NOUT), lambda i: (0, 0)),
            pl.BlockSpec((1, NOUT), lambda i: (0, 0)),
        ],
        out_specs=pl.BlockSpec((1, 4, rows, 4 * NOUT), lambda i: (i, 0, 0, 0)),
        out_shape=jax.ShapeDtypeStruct((nblk, 4, rows, 4 * NOUT), jnp.float32),
    )(hp, u1r, u2r, u1b, u2b4, d1wt, d1b, d2wt, d2b)


def _pool_matrix():
    pm = np.zeros((HH, 4 * RR), np.float32)
    for c4 in range(HH):
        for r in range(4):
            for dc in range(4):
                pm[c4, r * RR + c4 * 4 + dc] = 1.0 / 16.0
    return jnp.asarray(pm)


def _kron4(w):
    return jnp.kron(jnp.eye(4, dtype=jnp.float32), w)


def _tile4(b):
    return jnp.tile(b, (4,)).reshape(1, -1)


def _mpnn_block(node_p, edge_feats, src, dst3, zeros, kx, ks, w):
    (e1wt, e1b, e2wt, e2b, cb, wiht, whht, bih, bhh) = w
    e2wt = e2wt.astype(jnp.bfloat16)
    e2b = e2b.reshape(HH, HH).astype(jnp.bfloat16)
    cb4, wih4, whh4 = _tile4(cb), _kron4(wiht), _kron4(whht)
    bih4, bhh4 = _tile4(bih), _tile4(bhh)
    for _ in range(2):
        nsrc = _sc_gather_call()(node_p.reshape(VV, HH), src).reshape(EE // 4, 4 * HH)
        msg = _msg(edge_feats, nsrc, e1wt, e1b, e2wt, e2b, kx, ks)
        parts = _sc_scatter_call()(msg.reshape(EE, HH), dst3, zeros)
        node_p = _gru(parts.reshape(2 * VV // 4, 4 * HH), node_p,
                      cb4, wih4, whh4, bih4, bhh4)
    return node_p


def kernel(node_feats, edge_feats, edge_index,
           b0_p1W, b0_p1b, b0_p2W, b0_p2b, b0_e1W, b0_e1b, b0_e2W, b0_e2b,
           b0_cb, b0_Wih, b0_Whh, b0_bih, b0_bhh,
           b1_p1W, b1_p1b, b1_p2W, b1_p2b, b1_e1W, b1_e1b, b1_e2W, b1_e2b,
           b1_cb, b1_Wih, b1_Whh, b1_bih, b1_bhh,
           up1W, up1b, up2W, up2b, d1W, d1b, d2W, d2b):
    r2 = lambda v: v.reshape(1, -1)
    src = edge_index[0]
    dst3 = edge_index[1].reshape(NW, NCH, CH)
    zeros = jnp.zeros((VV, HH), jnp.float32)
    kx = jnp.kron(jnp.eye(HH, dtype=jnp.bfloat16), jnp.ones((1, HH), jnp.bfloat16))
    ks = jnp.kron(jnp.ones((HH, 1), jnp.bfloat16), jnp.eye(HH, dtype=jnp.bfloat16))

    x3 = node_feats.reshape(6 * HH, 4 * RR, CC)
    h0p = _encoder(x3, _pool_matrix(), b0_p1W.T, r2(b0_p1b), b0_p2W.T,
                   r2(b0_p2b))

    w0 = (b0_e1W.T, r2(b0_e1b), b0_e2W.T, r2(b0_e2b), r2(b0_cb),
          b0_Wih.T, b0_Whh.T, r2(b0_bih), r2(b0_bhh))
    node_p = _mpnn_block(h0p, edge_feats, src, dst3, zeros, kx, ks, w0)

    h1p = _mlp(node_p, _kron4(b1_p1W.T), _tile4(b1_p1b),
               _kron4(b1_p2W.T), _tile4(b1_p2b))
    w1 = (b1_e1W.T, r2(b1_e1b), b1_e2W.T, r2(b1_e2b), r2(b1_cb),
          b1_Wih.T, b1_Whh.T, r2(b1_bih), r2(b1_bhh))
    node_p = _mpnn_block(h1p, edge_feats, src, dst3, zeros, kx, ks, w1)

    u1r = up1W.transpose(2, 3, 0, 1).reshape(4 * HH, HH)
    u2r = up2W.transpose(0, 2, 3, 1).reshape(HH, 4 * HH)
    u2b4 = jnp.tile(up2b, (4,)).reshape(1, 4 * HH)
    o5 = _decoder(node_p, u1r, u2r, r2(up1b), u2b4, d1W.T, r2(d1b), d2W.T, r2(d2b))
    o = o5.reshape(24, 4, DEC_CELLS, HH, 4 * NOUT).transpose(0, 2, 1, 3, 4)
    return o.reshape(6 * RR * RR, NOUT)


# speedup vs baseline: 3.4061x; 1.0584x over previous
import functools

import jax
import jax.numpy as jnp
import numpy as np
from jax import lax
from jax.experimental import pallas as pl
from jax.experimental.pallas import tpu as pltpu
from jax.experimental.pallas import tpu_sc as plsc

RR = 128
CC = 128
HH = 32
EIN = 16
EHID = 64
NOUT = 3
VV = 6 * (RR // 4) ** 2
EE = VV * 16

NW = 32
EPW = EE // NW
CH = 128
NCH = EPW // CH
VS = VV // 16

@functools.cache
def _sc_gather_call():
    return pl.kernel(
        _sc_gather_body,
        out_type=jax.ShapeDtypeStruct((EE, HH), jnp.float32),
        mesh=plsc.VectorSubcoreMesh(core_axis_name="c", subcore_axis_name="s"),
        scratch_types=[
            pltpu.VMEM((EPW,), jnp.int32),
            pltpu.VMEM((EPW, HH), jnp.float32),
            pltpu.SemaphoreType.DMA,
        ],
        compiler_params=pltpu.CompilerParams(use_tc_tiling_on_sc=False),
    )


def _sc_gather_body(node_hbm, src_hbm, out_hbm, idx_v, rows_v, sem):
    wid = lax.axis_index("s") * 2 + lax.axis_index("c")
    base = wid * EPW
    pltpu.sync_copy(src_hbm.at[pl.ds(base, EPW)], idx_v)
    cps = []
    for j in range(NCH):
        cps.append(pltpu.async_copy(
            node_hbm.at[idx_v.at[pl.ds(j * CH, CH)]],
            rows_v.at[pl.ds(j * CH, CH)], sem))
    for cp in cps:
        cp.wait()
    pltpu.sync_copy(rows_v, out_hbm.at[pl.ds(base, EPW)])


@functools.cache
def _sc_scatter_call():
    return pl.kernel(
        _sc_scatter_body,
        out_type=jax.ShapeDtypeStruct((2 * VV, HH), jnp.float32),
        mesh=plsc.VectorSubcoreMesh(core_axis_name="c", subcore_axis_name="s"),
        scratch_types=[
            pltpu.VMEM((NCH, CH), jnp.int32),
            pltpu.VMEM((EPW, HH), jnp.float32),
            pltpu.VMEM_SHARED((VV, HH), jnp.float32),
        ],
        compiler_params=pltpu.CompilerParams(use_tc_tiling_on_sc=False),
    )


def _sc_scatter_body(msg_hbm, dst_hbm, zeros_hbm, out_hbm, idx_v, msg_v, acc):
    cid = lax.axis_index("c")
    sid = lax.axis_index("s")
    wid = sid * 2 + cid
    pltpu.sync_copy(zeros_hbm.at[pl.ds(sid * VS, VS)], acc.at[pl.ds(sid * VS, VS)])
    plsc.subcore_barrier()
    pltpu.sync_copy(dst_hbm.at[wid], idx_v)
    pltpu.sync_copy(msg_hbm.at[pl.ds(wid * EPW, EPW)], msg_v)
    for j in range(NCH):
        pltpu.sync_copy(msg_v.at[pl.ds(j * CH, CH)], acc.at[idx_v.at[j]], add=True)
    plsc.subcore_barrier()
    pltpu.sync_copy(acc.at[pl.ds(sid * VS, VS)],
                    out_hbm.at[pl.ds(cid * VV + sid * VS, VS)])


ENC_CELLS = 8


def _pack4(h):
    h3 = h.reshape(h.shape[0] // 4, 4, HH)
    return jnp.concatenate([h3[:, j, :] for j in range(4)], axis=1)


def _enc_body(x_ref, pm_ref, w1_ref, b1_ref, w2_ref, b2_ref, op_ref):
    for c in range(ENC_CELLS):
        x = x_ref[c]
        pooled = jnp.dot(pm_ref[...], x, preferred_element_type=jnp.float32)
        h = jnp.maximum(
            jnp.dot(pooled, w1_ref[...], preferred_element_type=jnp.float32)
            + b1_ref[...], 0.0)
        h = jnp.dot(h, w2_ref[...], preferred_element_type=jnp.float32) + b2_ref[...]
        op_ref[c * (HH // 4):(c + 1) * (HH // 4), :] = _pack4(h)


def _encoder(x3, pm, w1t, b1, w2t, b2):
    nblk = x3.shape[0] // ENC_CELLS
    return pl.pallas_call(
        _enc_body,
        grid=(nblk,),
        in_specs=[
            pl.BlockSpec((ENC_CELLS, 4 * RR, CC), lambda i: (i, 0, 0)),
            pl.BlockSpec((HH, 4 * RR), lambda i: (0, 0)),
            pl.BlockSpec((CC, HH), lambda i: (0, 0)),
            pl.BlockSpec((1, HH), lambda i: (0, 0)),
            pl.BlockSpec((HH, HH), lambda i: (0, 0)),
            pl.BlockSpec((1, HH), lambda i: (0, 0)),
        ],
        out_specs=pl.BlockSpec((ENC_CELLS * HH // 4, 4 * HH), lambda i: (i, 0)),
        out_shape=jax.ShapeDtypeStruct((VV // 4, 4 * HH), jnp.float32),
    )(x3, pm, w1t, b1, w2t, b2)


def _mlp_body(x_ref, w1_ref, b1_ref, w2_ref, b2_ref, op_ref):
    h = jnp.maximum(
        jnp.dot(x_ref[...], w1_ref[...], preferred_element_type=jnp.float32)
        + b1_ref[...], 0.0)
    op_ref[...] = jnp.dot(h, w2_ref[...], preferred_element_type=jnp.float32) \
        + b2_ref[...]


def _mlp(xp, w1t4, b14, w2t4, b24):
    return pl.pallas_call(
        _mlp_body,
        out_shape=jax.ShapeDtypeStruct((VV // 4, 4 * HH), jnp.float32),
    )(xp, w1t4, b14, w2t4, b24)


MSG_TILE = 2048


def _msg_body(ef_ref, ns_ref, e1w_ref, e1b_ref, e2w_ref, e2br_ref, kx_ref,
              ks_ref, o_ref):
    efp = ef_ref[...]
    nsp = ns_ref[...]
    for j in range(4):
        ef = efp[:, j * EIN:(j + 1) * EIN]
        ns = nsp[:, j * HH:(j + 1) * HH].astype(jnp.bfloat16)
        ew = jnp.maximum(
            jnp.dot(ef, e1w_ref[...], preferred_element_type=jnp.float32)
            + e1b_ref[...], 0.0)
        wm = jnp.dot(ew.astype(jnp.bfloat16), e2w_ref[...],
                     preferred_element_type=jnp.float32)
        x = jnp.dot(ns, kx_ref[...], preferred_element_type=jnp.float32)
        o_ref[:, j * HH:(j + 1) * HH] = (
            jnp.dot((x * wm).astype(jnp.bfloat16), ks_ref[...],
                    preferred_element_type=jnp.float32)
            + jnp.dot(ns, e2br_ref[...], preferred_element_type=jnp.float32))


def _msg(ef4, nsrc, e1wt, e1b, e2wt, e2b, kx, ks):
    return pl.pallas_call(
        _msg_body,
        grid=(EE // MSG_TILE,),
        in_specs=[
            pl.BlockSpec((MSG_TILE // 4, 4 * EIN), lambda i: (i, 0)),
            pl.BlockSpec((MSG_TILE // 4, 4 * HH), lambda i: (i, 0)),
            pl.BlockSpec((EIN, EHID), lambda i: (0, 0)),
            pl.BlockSpec((1, EHID), lambda i: (0, 0)),
            pl.BlockSpec((EHID, HH * HH), lambda i: (0, 0)),
            pl.BlockSpec((HH, HH), lambda i: (0, 0)),
            pl.BlockSpec((HH, HH * HH), lambda i: (0, 0)),
            pl.BlockSpec((HH * HH, HH), lambda i: (0, 0)),
        ],
        out_specs=pl.BlockSpec((MSG_TILE // 4, 4 * HH), lambda i: (i, 0)),
        out_shape=jax.ShapeDtypeStruct((EE // 4, 4 * HH), jnp.float32),
    )(ef4, nsrc, e1wt, e1b, e2wt, e2b, kx, ks)


def _gru_body(pp_ref, h_ref, cb4_ref, wih4_ref, whh4_ref, bih4_ref, bhh4_ref,
              op_ref):
    pp = pp_ref[...]
    m = jnp.maximum(pp[:VV // 4] + pp[VV // 4:] + cb4_ref[...], 0.0)
    h = h_ref[...]
    gi = jnp.dot(m, wih4_ref[...], preferred_element_type=jnp.float32) \
        + bih4_ref[...]
    gh = jnp.dot(h, whh4_ref[...], preferred_element_type=jnp.float32) \
        + bhh4_ref[...]
    for j in range(4):
        gij = gi[:, 3 * HH * j:3 * HH * (j + 1)]
        ghj = gh[:, 3 * HH * j:3 * HH * (j + 1)]
        r = jax.nn.sigmoid(gij[:, 0:HH] + ghj[:, 0:HH])
        z = jax.nn.sigmoid(gij[:, HH:2 * HH] + ghj[:, HH:2 * HH])
        n = jnp.tanh(gij[:, 2 * HH:3 * HH] + r * ghj[:, 2 * HH:3 * HH])
        hj = h[:, HH * j:HH * (j + 1)]
        op_ref[:, HH * j:HH * (j + 1)] = (1.0 - z) * n + z * hj


def _gru(pparts, hp, cb4, wih4, whh4, bih4, bhh4):
    return pl.pallas_call(
        _gru_body,
        out_shape=jax.ShapeDtypeStruct((VV // 4, 4 * HH), jnp.float32),
    )(pparts, hp, cb4, wih4, whh4, bih4, bhh4)


DEC_CELLS = 8


def _dec_body(x_ref, u1_ref, u2_ref, u1b_ref, u2b4_ref, d1_ref, d1b_ref,
              d2_ref, d2b_ref, o_ref):
    xp = x_ref[...]
    x = jnp.stack([xp[:, j * HH:(j + 1) * HH] for j in range(4)],
                  axis=1).reshape(DEC_CELLS * HH, HH)
    tt = jnp.dot(u1_ref[...], u2_ref[...], preferred_element_type=jnp.float32)
    b2v = jnp.dot(u1b_ref[...], u2_ref[...],
                  preferred_element_type=jnp.float32) + u2b4_ref[...]
    for p in range(4):
        cols = []
        for q in range(4):
            rblk = 2 * (p // 2) + (q // 2)
            cblk = 2 * (p % 2) + (q % 2)
            tpq = tt[rblk * HH:(rblk + 1) * HH, cblk * HH:(cblk + 1) * HH]
            y = jnp.dot(x, tpq, preferred_element_type=jnp.float32) \
                + b2v[:, cblk * HH:(cblk + 1) * HH]
            z = jnp.maximum(
                jnp.dot(y, d1_ref[...], preferred_element_type=jnp.float32)
                + d1b_ref[...], 0.0)
            cols.append(jnp.dot(z, d2_ref[...],
                                preferred_element_type=jnp.float32) + d2b_ref[...])
        o_ref[0, p] = jnp.concatenate(cols, axis=1)


def _decoder(hp, u1r, u2r, u1b, u2b4, d1wt, d1b, d2wt, d2b):
    rows = DEC_CELLS * HH
    nblk = hp.shape[0] // (rows // 4)
    return pl.pallas_call(
        _dec_body,
        grid=(nblk,),
        in_specs=[
            pl.BlockSpec((rows // 4, 4 * HH), lambda i: (i, 0)),
            pl.BlockSpec((4 * HH, HH), lambda i: (0, 0)),
            pl.BlockSpec((HH, 4 * HH), lambda i: (0, 0)),
            pl.BlockSpec((1, HH), lambda i: (0, 0)),
            pl.BlockSpec((1, 4 * HH), lambda i: (0, 0)),
            pl.BlockSpec((HH, 16), lambda i: (0, 0)),
            pl.BlockSpec((1, 16), lambda i: (0, 0)),
            pl.BlockSpec((16, NOUT), lambda i: (0, 0)),
            pl.BlockSpec((1, NOUT), lambda i: (0, 0)),
        ],
        out_specs=pl.BlockSpec((1, 4, rows, 4 * NOUT), lambda i: (i, 0, 0, 0)),
        out_shape=jax.ShapeDtypeStruct((nblk, 4, rows, 4 * NOUT), jnp.float32),
    )(hp, u1r, u2r, u1b, u2b4, d1wt, d1b, d2wt, d2b)


def _pool_matrix():
    pm = np.zeros((HH, 4 * RR), np.float32)
    for c4 in range(HH):
        for r in range(4):
            for dc in range(4):
                pm[c4, r * RR + c4 * 4 + dc] = 1.0 / 16.0
    return jnp.asarray(pm)


def _kron4(w):
    return jnp.kron(jnp.eye(4, dtype=jnp.float32), w)


def _tile4(b):
    return jnp.tile(b, (4,)).reshape(1, -1)


def _mpnn_block(node_p, edge_feats, src, dst3, zeros, kx, ks, w):
    (e1wt, e1b, e2wt, e2b, cb, wiht, whht, bih, bhh) = w
    e2wt = e2wt.astype(jnp.bfloat16)
    e2b = e2b.reshape(HH, HH).astype(jnp.bfloat16)
    cb4, wih4, whh4 = _tile4(cb), _kron4(wiht), _kron4(whht)
    bih4, bhh4 = _tile4(bih), _tile4(bhh)
    ef4 = edge_feats.reshape(EE // 4, 4 * EIN)
    for _ in range(2):
        nsrc = _sc_gather_call()(node_p.reshape(VV, HH), src).reshape(EE // 4, 4 * HH)
        msg = _msg(ef4, nsrc, e1wt, e1b, e2wt, e2b, kx, ks)
        parts = _sc_scatter_call()(msg.reshape(EE, HH), dst3, zeros)
        node_p = _gru(parts.reshape(2 * VV // 4, 4 * HH), node_p,
                      cb4, wih4, whh4, bih4, bhh4)
    return node_p


def kernel(node_feats, edge_feats, edge_index,
           b0_p1W, b0_p1b, b0_p2W, b0_p2b, b0_e1W, b0_e1b, b0_e2W, b0_e2b,
           b0_cb, b0_Wih, b0_Whh, b0_bih, b0_bhh,
           b1_p1W, b1_p1b, b1_p2W, b1_p2b, b1_e1W, b1_e1b, b1_e2W, b1_e2b,
           b1_cb, b1_Wih, b1_Whh, b1_bih, b1_bhh,
           up1W, up1b, up2W, up2b, d1W, d1b, d2W, d2b):
    r2 = lambda v: v.reshape(1, -1)
    src = edge_index[0]
    dst3 = edge_index[1].reshape(NW, NCH, CH)
    zeros = jnp.zeros((VV, HH), jnp.float32)
    kx = jnp.kron(jnp.eye(HH, dtype=jnp.bfloat16), jnp.ones((1, HH), jnp.bfloat16))
    ks = jnp.kron(jnp.ones((HH, 1), jnp.bfloat16), jnp.eye(HH, dtype=jnp.bfloat16))

    x3 = node_feats.reshape(6 * HH, 4 * RR, CC)
    h0p = _encoder(x3, _pool_matrix(), b0_p1W.T, r2(b0_p1b), b0_p2W.T,
                   r2(b0_p2b))

    w0 = (b0_e1W.T, r2(b0_e1b), b0_e2W.T, r2(b0_e2b), r2(b0_cb),
          b0_Wih.T, b0_Whh.T, r2(b0_bih), r2(b0_bhh))
    node_p = _mpnn_block(h0p, edge_feats, src, dst3, zeros, kx, ks, w0)

    h1p = _mlp(node_p, _kron4(b1_p1W.T), _tile4(b1_p1b),
               _kron4(b1_p2W.T), _tile4(b1_p2b))
    w1 = (b1_e1W.T, r2(b1_e1b), b1_e2W.T, r2(b1_e2b), r2(b1_cb),
          b1_Wih.T, b1_Whh.T, r2(b1_bih), r2(b1_bhh))
    node_p = _mpnn_block(h1p, edge_feats, src, dst3, zeros, kx, ks, w1)

    u1r = up1W.transpose(2, 3, 0, 1).reshape(4 * HH, HH)
    u2r = up2W.transpose(0, 2, 3, 1).reshape(HH, 4 * HH)
    u2b4 = jnp.tile(up2b, (4,)).reshape(1, 4 * HH)
    o5 = _decoder(node_p, u1r, u2r, r2(up1b), u2b4, d1W.T, r2(d1b), d2W.T, r2(d2b))
    o = o5.reshape(24, 4, DEC_CELLS, HH, 4 * NOUT).transpose(0, 2, 1, 3, 4)
    return o.reshape(6 * RR * RR, NOUT)


# speedup vs baseline: 3.6571x; 1.0737x over previous
import functools

import jax
import jax.numpy as jnp
import numpy as np
from jax import lax
from jax.experimental import pallas as pl
from jax.experimental.pallas import tpu as pltpu
from jax.experimental.pallas import tpu_sc as plsc

RR = 128
CC = 128
HH = 32
EIN = 16
EHID = 64
NOUT = 3
VV = 6 * (RR // 4) ** 2
EE = VV * 16

NW = 32
EPW = EE // NW
CH = 128
NCH = EPW // CH
VS = VV // 16

@functools.cache
def _sc_gather_call():
    return pl.kernel(
        _sc_gather_body,
        out_type=jax.ShapeDtypeStruct((EE, HH), jnp.float32),
        mesh=plsc.VectorSubcoreMesh(core_axis_name="c", subcore_axis_name="s"),
        scratch_types=[
            pltpu.VMEM((EPW,), jnp.int32),
            pltpu.VMEM((EPW, HH), jnp.float32),
            pltpu.SemaphoreType.DMA,
        ],
        compiler_params=pltpu.CompilerParams(use_tc_tiling_on_sc=False),
    )


def _sc_gather_body(node_hbm, src_hbm, out_hbm, idx_v, rows_v, sem):
    wid = lax.axis_index("s") * 2 + lax.axis_index("c")
    base = wid * EPW
    pltpu.sync_copy(src_hbm.at[pl.ds(base, EPW)], idx_v)
    cps = []
    for j in range(NCH):
        cps.append(pltpu.async_copy(
            node_hbm.at[idx_v.at[pl.ds(j * CH, CH)]],
            rows_v.at[pl.ds(j * CH, CH)], sem))
    for cp in cps:
        cp.wait()
    pltpu.sync_copy(rows_v, out_hbm.at[pl.ds(base, EPW)])


@functools.cache
def _sc_scatter_call():
    return pl.kernel(
        _sc_scatter_body,
        out_type=jax.ShapeDtypeStruct((2 * VV, HH), jnp.float32),
        mesh=plsc.VectorSubcoreMesh(core_axis_name="c", subcore_axis_name="s"),
        scratch_types=[
            pltpu.VMEM((NCH, CH), jnp.int32),
            pltpu.VMEM((EPW, HH), jnp.float32),
            pltpu.VMEM_SHARED((VV, HH), jnp.float32),
        ],
        compiler_params=pltpu.CompilerParams(use_tc_tiling_on_sc=False),
    )


def _sc_scatter_body(msg_hbm, dst_hbm, zeros_hbm, out_hbm, idx_v, msg_v, acc):
    cid = lax.axis_index("c")
    sid = lax.axis_index("s")
    wid = sid * 2 + cid
    pltpu.sync_copy(zeros_hbm.at[pl.ds(sid * VS, VS)], acc.at[pl.ds(sid * VS, VS)])
    plsc.subcore_barrier()
    pltpu.sync_copy(dst_hbm.at[wid], idx_v)
    pltpu.sync_copy(msg_hbm.at[pl.ds(wid * EPW, EPW)], msg_v)
    for j in range(NCH):
        pltpu.sync_copy(msg_v.at[pl.ds(j * CH, CH)], acc.at[idx_v.at[j]], add=True)
    plsc.subcore_barrier()
    pltpu.sync_copy(acc.at[pl.ds(sid * VS, VS)],
                    out_hbm.at[pl.ds(cid * VV + sid * VS, VS)])


ENC_CELLS = 8


def _pack4(h):
    h3 = h.reshape(h.shape[0] // 4, 4, HH)
    return jnp.concatenate([h3[:, j, :] for j in range(4)], axis=1)


def _enc_body(x_ref, pm_ref, w1_ref, b1_ref, w2_ref, b2_ref, op_ref):
    for c in range(ENC_CELLS):
        x = x_ref[c]
        pooled = jnp.dot(pm_ref[...], x, preferred_element_type=jnp.float32)
        h = jnp.maximum(
            jnp.dot(pooled, w1_ref[...], preferred_element_type=jnp.float32)
            + b1_ref[...], 0.0)
        h = jnp.dot(h, w2_ref[...], preferred_element_type=jnp.float32) + b2_ref[...]
        op_ref[c * (HH // 4):(c + 1) * (HH // 4), :] = _pack4(h)


def _encoder(x3, pm, w1t, b1, w2t, b2):
    nblk = x3.shape[0] // ENC_CELLS
    return pl.pallas_call(
        _enc_body,
        grid=(nblk,),
        in_specs=[
            pl.BlockSpec((ENC_CELLS, 4 * RR, CC), lambda i: (i, 0, 0)),
            pl.BlockSpec((HH, 4 * RR), lambda i: (0, 0)),
            pl.BlockSpec((CC, HH), lambda i: (0, 0)),
            pl.BlockSpec((1, HH), lambda i: (0, 0)),
            pl.BlockSpec((HH, HH), lambda i: (0, 0)),
            pl.BlockSpec((1, HH), lambda i: (0, 0)),
        ],
        out_specs=pl.BlockSpec((ENC_CELLS * HH // 4, 4 * HH), lambda i: (i, 0)),
        out_shape=jax.ShapeDtypeStruct((VV // 4, 4 * HH), jnp.float32),
    )(x3, pm, w1t, b1, w2t, b2)


def _mlp_body(x_ref, w1_ref, b1_ref, w2_ref, b2_ref, op_ref):
    h = jnp.maximum(
        jnp.dot(x_ref[...], w1_ref[...], preferred_element_type=jnp.float32)
        + b1_ref[...], 0.0)
    op_ref[...] = jnp.dot(h, w2_ref[...], preferred_element_type=jnp.float32) \
        + b2_ref[...]


def _mlp(xp, w1t4, b14, w2t4, b24):
    return pl.pallas_call(
        _mlp_body,
        out_shape=jax.ShapeDtypeStruct((VV // 4, 4 * HH), jnp.float32),
    )(xp, w1t4, b14, w2t4, b24)


MSG_TILE = 4096


def _msg_body(ef_ref, ns_ref, e1w_ref, e1b_ref, e2w_ref, e2br_ref, kx_ref,
              ks_ref, o_ref):
    efp = ef_ref[...]
    nsp = ns_ref[...]
    for j in range(4):
        ef = efp[:, j * EIN:(j + 1) * EIN]
        ns = nsp[:, j * HH:(j + 1) * HH].astype(jnp.bfloat16)
        ew = jnp.maximum(
            jnp.dot(ef, e1w_ref[...], preferred_element_type=jnp.float32)
            + e1b_ref[...], 0.0)
        wm = jnp.dot(ew.astype(jnp.bfloat16), e2w_ref[...],
                     preferred_element_type=jnp.float32)
        x = jnp.dot(ns, kx_ref[...], preferred_element_type=jnp.float32)
        o_ref[:, j * HH:(j + 1) * HH] = (
            jnp.dot((x * wm).astype(jnp.bfloat16), ks_ref[...],
                    preferred_element_type=jnp.float32)
            + jnp.dot(ns, e2br_ref[...], preferred_element_type=jnp.float32))


def _msg(ef4, nsrc, e1wt, e1b, e2wt, e2b, kx, ks):
    return pl.pallas_call(
        _msg_body,
        grid=(EE // MSG_TILE,),
        in_specs=[
            pl.BlockSpec((MSG_TILE // 4, 4 * EIN), lambda i: (i, 0)),
            pl.BlockSpec((MSG_TILE // 4, 4 * HH), lambda i: (i, 0)),
            pl.BlockSpec((EIN, EHID), lambda i: (0, 0)),
            pl.BlockSpec((1, EHID), lambda i: (0, 0)),
            pl.BlockSpec((EHID, HH * HH), lambda i: (0, 0)),
            pl.BlockSpec((HH, HH), lambda i: (0, 0)),
            pl.BlockSpec((HH, HH * HH), lambda i: (0, 0)),
            pl.BlockSpec((HH * HH, HH), lambda i: (0, 0)),
        ],
        out_specs=pl.BlockSpec((MSG_TILE // 4, 4 * HH), lambda i: (i, 0)),
        out_shape=jax.ShapeDtypeStruct((EE // 4, 4 * HH), jnp.float32),
    )(ef4, nsrc, e1wt, e1b, e2wt, e2b, kx, ks)


def _gru_body(pp_ref, h_ref, cb4_ref, wih4_ref, whh4_ref, bih4_ref, bhh4_ref,
              op_ref):
    pp = pp_ref[...]
    m = jnp.maximum(pp[:VV // 4] + pp[VV // 4:] + cb4_ref[...], 0.0)
    h = h_ref[...]
    gi = jnp.dot(m, wih4_ref[...], preferred_element_type=jnp.float32) \
        + bih4_ref[...]
    gh = jnp.dot(h, whh4_ref[...], preferred_element_type=jnp.float32) \
        + bhh4_ref[...]
    for j in range(4):
        gij = gi[:, 3 * HH * j:3 * HH * (j + 1)]
        ghj = gh[:, 3 * HH * j:3 * HH * (j + 1)]
        r = jax.nn.sigmoid(gij[:, 0:HH] + ghj[:, 0:HH])
        z = jax.nn.sigmoid(gij[:, HH:2 * HH] + ghj[:, HH:2 * HH])
        n = jnp.tanh(gij[:, 2 * HH:3 * HH] + r * ghj[:, 2 * HH:3 * HH])
        hj = h[:, HH * j:HH * (j + 1)]
        op_ref[:, HH * j:HH * (j + 1)] = (1.0 - z) * n + z * hj


def _gru(pparts, hp, cb4, wih4, whh4, bih4, bhh4):
    return pl.pallas_call(
        _gru_body,
        out_shape=jax.ShapeDtypeStruct((VV // 4, 4 * HH), jnp.float32),
    )(pparts, hp, cb4, wih4, whh4, bih4, bhh4)


DEC_CELLS = 8


def _dec_body(x_ref, u1_ref, u2_ref, u1b_ref, u2b4_ref, d1_ref, d1b_ref,
              d2_ref, d2b_ref, o_ref):
    xp = x_ref[...]
    x = jnp.stack([xp[:, j * HH:(j + 1) * HH] for j in range(4)],
                  axis=1).reshape(DEC_CELLS * HH, HH)
    tt = jnp.dot(u1_ref[...], u2_ref[...], preferred_element_type=jnp.float32)
    b2v = jnp.dot(u1b_ref[...], u2_ref[...],
                  preferred_element_type=jnp.float32) + u2b4_ref[...]
    for p in range(4):
        cols = []
        for q in range(4):
            rblk = 2 * (p // 2) + (q // 2)
            cblk = 2 * (p % 2) + (q % 2)
            tpq = tt[rblk * HH:(rblk + 1) * HH, cblk * HH:(cblk + 1) * HH]
            y = jnp.dot(x, tpq, preferred_element_type=jnp.float32) \
                + b2v[:, cblk * HH:(cblk + 1) * HH]
            z = jnp.maximum(
                jnp.dot(y, d1_ref[...], preferred_element_type=jnp.float32)
                + d1b_ref[...], 0.0)
            cols.append(jnp.dot(z, d2_ref[...],
                                preferred_element_type=jnp.float32) + d2b_ref[...])
        o_ref[0, p] = jnp.concatenate(cols, axis=1)


def _decoder(hp, u1r, u2r, u1b, u2b4, d1wt, d1b, d2wt, d2b):
    rows = DEC_CELLS * HH
    nblk = hp.shape[0] // (rows // 4)
    return pl.pallas_call(
        _dec_body,
        grid=(nblk,),
        in_specs=[
            pl.BlockSpec((rows // 4, 4 * HH), lambda i: (i, 0)),
            pl.BlockSpec((4 * HH, HH), lambda i: (0, 0)),
            pl.BlockSpec((HH, 4 * HH), lambda i: (0, 0)),
            pl.BlockSpec((1, HH), lambda i: (0, 0)),
            pl.BlockSpec((1, 4 * HH), lambda i: (0, 0)),
            pl.BlockSpec((HH, 16), lambda i: (0, 0)),
            pl.BlockSpec((1, 16), lambda i: (0, 0)),
            pl.BlockSpec((16, NOUT), lambda i: (0, 0)),
            pl.BlockSpec((1, NOUT), lambda i: (0, 0)),
        ],
        out_specs=pl.BlockSpec((1, 4, rows, 4 * NOUT), lambda i: (i, 0, 0, 0)),
        out_shape=jax.ShapeDtypeStruct((nblk, 4, rows, 4 * NOUT), jnp.float32),
    )(hp, u1r, u2r, u1b, u2b4, d1wt, d1b, d2wt, d2b)


def _pool_matrix():
    pm = np.zeros((HH, 4 * RR), np.float32)
    for c4 in range(HH):
        for r in range(4):
            for dc in range(4):
                pm[c4, r * RR + c4 * 4 + dc] = 1.0 / 16.0
    return jnp.asarray(pm)


def _kron4(w):
    return jnp.kron(jnp.eye(4, dtype=jnp.float32), w)


def _tile4(b):
    return jnp.tile(b, (4,)).reshape(1, -1)


def _mpnn_block(node_p, edge_feats, src, dst3, zeros, kx, ks, w):
    (e1wt, e1b, e2wt, e2b, cb, wiht, whht, bih, bhh) = w
    e2wt = e2wt.astype(jnp.bfloat16)
    e2b = e2b.reshape(HH, HH).astype(jnp.bfloat16)
    cb4, wih4, whh4 = _tile4(cb), _kron4(wiht), _kron4(whht)
    bih4, bhh4 = _tile4(bih), _tile4(bhh)
    ef4 = edge_feats.reshape(EE // 4, 4 * EIN)
    for _ in range(2):
        nsrc = _sc_gather_call()(node_p.reshape(VV, HH), src).reshape(EE // 4, 4 * HH)
        msg = _msg(ef4, nsrc, e1wt, e1b, e2wt, e2b, kx, ks)
        parts = _sc_scatter_call()(msg.reshape(EE, HH), dst3, zeros)
        node_p = _gru(parts.reshape(2 * VV // 4, 4 * HH), node_p,
                      cb4, wih4, whh4, bih4, bhh4)
    return node_p


def kernel(node_feats, edge_feats, edge_index,
           b0_p1W, b0_p1b, b0_p2W, b0_p2b, b0_e1W, b0_e1b, b0_e2W, b0_e2b,
           b0_cb, b0_Wih, b0_Whh, b0_bih, b0_bhh,
           b1_p1W, b1_p1b, b1_p2W, b1_p2b, b1_e1W, b1_e1b, b1_e2W, b1_e2b,
           b1_cb, b1_Wih, b1_Whh, b1_bih, b1_bhh,
           up1W, up1b, up2W, up2b, d1W, d1b, d2W, d2b):
    r2 = lambda v: v.reshape(1, -1)
    src = edge_index[0]
    dst3 = edge_index[1].reshape(NW, NCH, CH)
    zeros = jnp.zeros((VV, HH), jnp.float32)
    kx = jnp.kron(jnp.eye(HH, dtype=jnp.bfloat16), jnp.ones((1, HH), jnp.bfloat16))
    ks = jnp.kron(jnp.ones((HH, 1), jnp.bfloat16), jnp.eye(HH, dtype=jnp.bfloat16))

    x3 = node_feats.reshape(6 * HH, 4 * RR, CC)
    h0p = _encoder(x3, _pool_matrix(), b0_p1W.T, r2(b0_p1b), b0_p2W.T,
                   r2(b0_p2b))

    w0 = (b0_e1W.T, r2(b0_e1b), b0_e2W.T, r2(b0_e2b), r2(b0_cb),
          b0_Wih.T, b0_Whh.T, r2(b0_bih), r2(b0_bhh))
    node_p = _mpnn_block(h0p, edge_feats, src, dst3, zeros, kx, ks, w0)

    h1p = _mlp(node_p, _kron4(b1_p1W.T), _tile4(b1_p1b),
               _kron4(b1_p2W.T), _tile4(b1_p2b))
    w1 = (b1_e1W.T, r2(b1_e1b), b1_e2W.T, r2(b1_e2b), r2(b1_cb),
          b1_Wih.T, b1_Whh.T, r2(b1_bih), r2(b1_bhh))
    node_p = _mpnn_block(h1p, edge_feats, src, dst3, zeros, kx, ks, w1)

    u1r = up1W.transpose(2, 3, 0, 1).reshape(4 * HH, HH)
    u2r = up2W.transpose(0, 2, 3, 1).reshape(HH, 4 * HH)
    u2b4 = jnp.tile(up2b, (4,)).reshape(1, 4 * HH)
    o5 = _decoder(node_p, u1r, u2r, r2(up1b), u2b4, d1W.T, r2(d1b), d2W.T, r2(d2b))
    o = o5.reshape(24, 4, DEC_CELLS, HH, 4 * NOUT).transpose(0, 2, 1, 3, 4)
    return o.reshape(6 * RR * RR, NOUT)
